# baseline (speedup 1.0000x reference)
import functools

import jax
import jax.numpy as jnp
from jax import lax
from jax.experimental import pallas as pl
from jax.experimental.pallas import tpu as pltpu
from jax.experimental.pallas import tpu_sc as plsc

H = 128
NW = 32
LANES = 16


def _ptree(vals):
    vals = list(vals)
    while len(vals) > 1:
        nxt = [a + b for a, b in zip(vals[0::2], vals[1::2])]
        if len(vals) % 2:
            nxt.append(vals[-1])
        vals = nxt
    return vals[0]


def _wid():
    return lax.axis_index("s") * 2 + lax.axis_index("c")


def _mesh():
    return plsc.VectorSubcoreMesh(core_axis_name="c", subcore_axis_name="s")


def _mm0_body(fb, wi, wh, inp_o, hm_o):
    inp = jnp.dot(fb[...], wi[...], preferred_element_type=jnp.float32)
    inp_o[...] = inp
    hm_o[...] = jnp.dot(jnp.maximum(inp, 0.0), wh[...],
                        preferred_element_type=jnp.float32)


def _tc_mm0(f_bonds, W_i, W_h):
    nb = f_bonds.shape[0]
    B = 4000
    return pl.pallas_call(
        _mm0_body,
        grid=(nb // B,),
        in_specs=[
            pl.BlockSpec((B, f_bonds.shape[1]), lambda i: (i, 0)),
            pl.BlockSpec(W_i.shape, lambda i: (0, 0)),
            pl.BlockSpec(W_h.shape, lambda i: (0, 0)),
        ],
        out_specs=[
            pl.BlockSpec((B, H), lambda i: (i, 0)),
            pl.BlockSpec((B, H), lambda i: (i, 0)),
        ],
        out_shape=[
            jax.ShapeDtypeStruct((nb, H), jnp.float32),
            jax.ShapeDtypeStruct((nb, H), jnp.float32),
        ],
    )(f_bonds, W_i, W_h)


def _mm1_body(inp, g, wh, hm_o):
    m = jnp.maximum(inp[...] + g[...], 0.0)
    hm_o[...] = jnp.dot(m, wh[...], preferred_element_type=jnp.float32)


def _tc_mm1(inp, g, W_h, rows=None):
    nb = inp.shape[0] if rows is None else rows
    B = 4000
    return pl.pallas_call(
        _mm1_body,
        grid=(nb // B,),
        in_specs=[
            pl.BlockSpec((B, H), lambda i: (i, 0)),
            pl.BlockSpec((B, H), lambda i: (i, 0)),
            pl.BlockSpec(W_h.shape, lambda i: (0, 0)),
        ],
        out_specs=pl.BlockSpec((B, H), lambda i: (i, 0)),
        out_shape=jax.ShapeDtypeStruct((nb, H), jnp.float32),
    )(inp, g, W_h)


def _sc_gathersum(hm, a2b_flat):
    P = 2
    nrows = a2b_flat.shape[0]
    n_at = nrows // 16
    CA = 16
    RPC = CA * 16
    n_chunks = n_at // CA
    NC = -(-n_chunks // NW)
    NC = -(-NC // P) * P
    SLAB = NC * RPC

    @functools.partial(
        pl.kernel,
        out_type=jax.ShapeDtypeStruct((n_at, H), jnp.float32),
        mesh=_mesh(),
        scratch_types=(
            [pltpu.VMEM((SLAB,), jnp.int32)]
            + [pltpu.VMEM((RPC, H), jnp.float32) for _ in range(P)]
            + [pltpu.VMEM((CA, H), jnp.float32) for _ in range(P)]
            + [pltpu.SemaphoreType.DMA for _ in range(2 * P)]
        ),
    )
    def k(hm_hbm, idx_hbm, out_hbm, idx_s, *bufs):
        rows = bufs[0:P]
        outs = bufs[P:2 * P]
        srs = bufs[2 * P:3 * P]
        sos = bufs[3 * P:4 * P]
        w = _wid()
        c0 = (w * (n_chunks - NC)) // (NW - 1)

        pltpu.sync_copy(idx_hbm.at[pl.ds(c0 * RPC, SLAB)], idx_s)
        for p in range(P):
            pltpu.async_copy(
                hm_hbm.at[idx_s.at[pl.ds(p * RPC, RPC)]], rows[p], srs[p])

        def body(cg, _):
            for p in range(P):
                ci = P * cg + p
                pltpu.make_async_copy(
                    hm_hbm.at[idx_s.at[pl.ds(0, RPC)]], rows[p],
                    srs[p]).wait()

                @pl.when(cg > 0)
                def _():
                    pltpu.make_async_copy(
                        outs[p], out_hbm.at[pl.ds(0, CA)], sos[p]).wait()

                for a in range(CA):
                    for s in range(H // LANES):
                        sl = pl.ds(s * LANES, LANES)
                        acc = rows[p][a * 16, sl]
                        for j in range(1, 16):
                            acc = acc + rows[p][a * 16 + j, sl]
                        outs[p][a, sl] = acc
                pltpu.async_copy(
                    outs[p], out_hbm.at[pl.ds((c0 + ci) * CA, CA)], sos[p])

                @pl.when(ci + P < NC)
                def _():
                    pltpu.async_copy(
                        hm_hbm.at[idx_s.at[pl.ds((ci + P) * RPC, RPC)]],
                        rows[p], srs[p])
            return 0

        lax.fori_loop(0, NC // P, body, 0)
        for p in range(P):
            pltpu.make_async_copy(
                outs[p], out_hbm.at[pl.ds(0, CA)], sos[p]).wait()

    return k(hm, a2b_flat)


def _sc_combine(ahm, hm, b2a, b2revb, inp=None, lo=0, hi=None):
    nb = hm.shape[0]
    if hi is None:
        hi = nb
    span = hi - lo
    per_w = span // NW
    R = 64
    P = 2
    n_full = per_w // R
    NC = -(-(n_full + 1) // P) * P
    with_inp = inp is not None

    scratch = (
        [pltpu.VMEM((per_w,), jnp.int32) for _ in range(2)]
        + [pltpu.VMEM((R, H), jnp.float32) for _ in range(3 * P)]
        + [pltpu.SemaphoreType.DMA for _ in range(2 * P)]
    )
    if with_inp:
        scratch += (
            [pltpu.VMEM((R, H), jnp.float32) for _ in range(P)]
            + [pltpu.SemaphoreType.DMA for _ in range(P)]
        )

    def body(ahm_hbm, hm_hbm, b2a_hbm, b2revb_hbm, *rest):
        if with_inp:
            inp_hbm, out_hbm = rest[0], rest[1]
            bufs = rest[2:]
        else:
            out_hbm = rest[0]
            bufs = rest[1:]
        ia_s, ib_s = bufs[0], bufs[1]
        ras = bufs[2:2 + P]
        rbs = bufs[2 + P:2 + 2 * P]
        ous = bufs[2 + 2 * P:2 + 3 * P]
        srs = bufs[2 + 3 * P:2 + 4 * P]
        sos = bufs[2 + 4 * P:2 + 5 * P]
        if with_inp:
            ris = bufs[2 + 5 * P:2 + 6 * P]
            sis = bufs[2 + 6 * P:2 + 7 * P]
        w = _wid()
        b0 = lo + w * per_w

        def cstart(ci):
            return jnp.where(ci < n_full, ci * R, per_w - (NC - ci) * R)

        def fetch(ci, p):
            st = cstart(ci)
            pltpu.async_copy(ahm_hbm.at[ib_s.at[pl.ds(st, R)]],
                             ras[p], srs[p])
            pltpu.async_copy(hm_hbm.at[ia_s.at[pl.ds(st, R)]],
                             rbs[p], srs[p])
            if with_inp:
                pltpu.async_copy(inp_hbm.at[pl.ds(b0 + st, R)],
                                 ris[p], sis[p])

        pltpu.sync_copy(b2a_hbm.at[pl.ds(b0, per_w)], ib_s)
        pltpu.sync_copy(b2revb_hbm.at[pl.ds(b0, per_w)], ia_s)
        for p in range(P):
            fetch(p, p)

        def chunk(cg, _):
            for p in range(P):
                ci = P * cg + p
                st = cstart(ci)
                for _ in range(2):
                    pltpu.make_async_copy(
                        hm_hbm.at[ia_s.at[pl.ds(0, R)]], rbs[p],
                        srs[p]).wait()
                if with_inp:
                    pltpu.make_async_copy(
                        inp_hbm.at[pl.ds(0, R)], ris[p], sis[p]).wait()

                @pl.when(cg > 0)
                def _():
                    pltpu.make_async_copy(
                        ous[p], out_hbm.at[pl.ds(0, R)], sos[p]).wait()

                def row(r, _):
                    for s in range(H // LANES):
                        sl = pl.ds(s * LANES, LANES)
                        x = ras[p][r, sl] - rbs[p][r, sl]
                        if with_inp:
                            x = jnp.maximum(x + ris[p][r, sl], 0.0)
                        ous[p][r, sl] = x
                    return 0

                lax.fori_loop(0, R, row, 0)
                pltpu.async_copy(
                    ous[p], out_hbm.at[pl.ds(b0 + st, R)], sos[p])

                @pl.when(ci + P < NC)
                def _():
                    fetch(ci + P, p)
            return 0

        lax.fori_loop(0, NC // P, chunk, 0)
        for p in range(P):
            pltpu.make_async_copy(
                ous[p], out_hbm.at[pl.ds(0, R)], sos[p]).wait()

    kern = functools.partial(
        pl.kernel,
        out_type=jax.ShapeDtypeStruct((span, H), jnp.float32),
        mesh=_mesh(),
        scratch_types=scratch,
    )(body)
    if with_inp:
        return kern(ahm, hm, b2a, b2revb, inp)
    return kern(ahm, hm, b2a, b2revb)


def _sc_final(inp, ahm, hm, a2b_flat, b2a, b2revb):
    P = 2
    nrows = a2b_flat.shape[0]
    n_at = nrows // 16
    CA = 4
    RPC = CA * 16
    n_chunks = n_at // CA
    NC = -(-n_chunks // NW)
    NC = -(-NC // P) * P
    SLAB = NC * RPC

    @functools.partial(
        pl.kernel,
        out_type=jax.ShapeDtypeStruct((n_at, H), jnp.float32),
        mesh=_mesh(),
        scratch_types=(
            [pltpu.VMEM((SLAB,), jnp.int32)]
            + [pltpu.VMEM((RPC,), jnp.int32) for _ in range(2 * P)]
            + [pltpu.VMEM((RPC, H), jnp.float32) for _ in range(3 * P)]
            + [pltpu.VMEM((CA, H), jnp.float32) for _ in range(P)]
            + [pltpu.SemaphoreType.DMA for _ in range(3 * P)]
        ),
    )
    def k(inp_hbm, ahm_hbm, hm_hbm, a2b_hbm, b2a_hbm, b2revb_hbm, out_hbm,
          slab, *bufs):
        ias = bufs[0:P]
        ibs = bufs[P:2 * P]
        ris = bufs[2 * P:3 * P]
        ras = bufs[3 * P:4 * P]
        rbs = bufs[4 * P:5 * P]
        ous = bufs[5 * P:6 * P]
        sis = bufs[6 * P:7 * P]
        srs = bufs[7 * P:8 * P]
        sos = bufs[8 * P:9 * P]
        w = _wid()
        c0 = (w * (n_chunks - NC)) // (NW - 1)

        pltpu.sync_copy(a2b_hbm.at[pl.ds(c0 * RPC, SLAB)], slab)

        def fetch_idx(ci, p):
            sl = slab.at[pl.ds(ci * RPC, RPC)]
            pltpu.async_copy(b2a_hbm.at[sl], ias[p], sis[p])
            pltpu.async_copy(b2revb_hbm.at[sl], ibs[p], sis[p])

        def fetch_rows(ci, p):
            pltpu.async_copy(inp_hbm.at[slab.at[pl.ds(ci * RPC, RPC)]],
                             ris[p], srs[p])
            pltpu.async_copy(ahm_hbm.at[ias[p]], ras[p], srs[p])
            pltpu.async_copy(hm_hbm.at[ibs[p]], rbs[p], srs[p])

        def wait_idx(p):
            for _ in range(2):
                pltpu.make_async_copy(b2a_hbm.at[slab.at[pl.ds(0, RPC)]],
                                      ias[p], sis[p]).wait()

        def wait_rows(p):
            for _ in range(3):
                pltpu.make_async_copy(hm_hbm.at[ias[p]], rbs[p],
                                      srs[p]).wait()

        fetch_idx(0, 0)
        fetch_idx(1, 1)
        wait_idx(0)
        fetch_rows(0, 0)

        def body(cg, _):
            for p in range(P):
                c = P * cg + p
                pn = (p + 1) % P

                @pl.when(c + 1 < NC)
                def _():
                    wait_idx(pn)
                    fetch_rows(c + 1, pn)

                wait_rows(p)

                @pl.when(c + 2 < NC)
                def _():
                    fetch_idx(c + 2, p)

                @pl.when(cg > 0)
                def _():
                    pltpu.make_async_copy(
                        ous[p], out_hbm.at[pl.ds(0, CA)], sos[p]).wait()

                for a in range(CA):
                    for s in range(H // LANES):
                        sl = pl.ds(s * LANES, LANES)
                        r0 = a * 16
                        acc = jnp.maximum(
                            ris[p][r0, sl] + ras[p][r0, sl] - rbs[p][r0, sl],
                            0.0)
                        for j in range(1, 16):
                            r = r0 + j
                            acc = acc + jnp.maximum(
                                ris[p][r, sl] + ras[p][r, sl] - rbs[p][r, sl],
                                0.0)
                        ous[p][a, sl] = acc
                pltpu.async_copy(
                    ous[p], out_hbm.at[pl.ds((c0 + c) * CA, CA)], sos[p])
            return 0

        lax.fori_loop(0, NC // P, body, 0)
        for p in range(P):
            pltpu.make_async_copy(
                ous[p], out_hbm.at[pl.ds(0, CA)], sos[p]).wait()

    return k(inp, ahm, hm, a2b_flat, b2a, b2revb)


def _readout_body(fa, amr, segr, woa, wom, bor, out_ref, acc, cnt):
    i = pl.program_id(0)
    npg = pl.num_programs(0)

    @pl.when(i == 0)
    def _():
        acc[...] = jnp.zeros_like(acc)
        cnt[...] = jnp.zeros_like(cnt)

    ah = jnp.dot(fa[...], woa[...], preferred_element_type=jnp.float32)
    ah = ah + jnp.dot(amr[...], wom[...], preferred_element_type=jnp.float32)
    ah = jnp.maximum(ah + bor[...], 0.0)

    s = segr[0]
    A = s.shape[1]
    n_mols = acc.shape[0]
    MC = 500
    for h in range(n_mols // MC):
        iota = lax.broadcasted_iota(jnp.int32, (MC, A), 0) + h * MC
        ohT = (jnp.broadcast_to(s, (MC, A)) == iota).astype(jnp.float32)
        acc[pl.ds(h * MC, MC), :] += jnp.dot(
            ohT, ah, preferred_element_type=jnp.float32)
        cnt[pl.ds(h * MC, MC), :] += jnp.sum(ohT, axis=1, keepdims=True)

    @pl.when(i == npg - 1)
    def _():
        out_ref[...] = acc[...] / jnp.maximum(cnt[...], 1.0)


def _tc_readout(f_atoms, am, seg, W_o, b_o, n_mols=2000):
    na = f_atoms.shape[0]
    A = 1000
    seg3 = seg.reshape(na // A, 1, A)
    woa = W_o[:H]
    wom = W_o[H:]
    bor = b_o.reshape(1, H)
    return pl.pallas_call(
        _readout_body,
        grid=(na // A,),
        in_specs=[
            pl.BlockSpec((A, H), lambda i: (i, 0)),
            pl.BlockSpec((A, H), lambda i: (i, 0)),
            pl.BlockSpec((1, 1, A), lambda i: (i, 0, 0)),
            pl.BlockSpec((H, H), lambda i: (0, 0)),
            pl.BlockSpec((H, H), lambda i: (0, 0)),
            pl.BlockSpec((1, H), lambda i: (0, 0)),
        ],
        out_specs=pl.BlockSpec((n_mols, H), lambda i: (0, 0)),
        out_shape=jax.ShapeDtypeStruct((n_mols, H), jnp.float32),
        scratch_shapes=[
            pltpu.VMEM((n_mols, H), jnp.float32),
            pltpu.VMEM((n_mols, 1), jnp.float32),
        ],
        compiler_params=pltpu.CompilerParams(
            dimension_semantics=("arbitrary",)),
    )(f_atoms, am, seg3, woa, wom, bor)


def kernel(f_atoms, f_bonds, a2b, b2a, b2revb, seg, W_i, W_h, W_o, b_o):
    a2b_flat = a2b.reshape(-1)
    nb = f_bonds.shape[0]
    half = nb // 2

    inp, hm = _tc_mm0(f_bonds, W_i, W_h)
    ahm0 = _sc_gathersum(hm, a2b_flat)
    g0 = _sc_combine(ahm0, hm, b2a, b2revb)
    hm1 = _tc_mm1(inp, g0, W_h)
    ahm1 = _sc_gathersum(hm1, a2b_flat)
    msg2 = _sc_combine(ahm1, hm1, b2a, b2revb, inp=inp)
    am2 = _sc_gathersum(msg2, a2b_flat)
    return _tc_readout(f_atoms, am2, seg, W_o, b_o)


# speedup vs baseline: 1.0031x; 1.0031x over previous
import functools

import jax
import jax.numpy as jnp
from jax import lax
from jax.experimental import pallas as pl
from jax.experimental.pallas import tpu as pltpu
from jax.experimental.pallas import tpu_sc as plsc

H = 128
NW = 32
LANES = 16


def _ptree(vals):
    vals = list(vals)
    while len(vals) > 1:
        nxt = [a + b for a, b in zip(vals[0::2], vals[1::2])]
        if len(vals) % 2:
            nxt.append(vals[-1])
        vals = nxt
    return vals[0]


def _wid():
    return lax.axis_index("s") * 2 + lax.axis_index("c")


def _mesh():
    return plsc.VectorSubcoreMesh(core_axis_name="c", subcore_axis_name="s")


def _mm0_body(fb, wi, wh, inp_o, hm_o):
    inp = jnp.dot(fb[...], wi[...], preferred_element_type=jnp.float32)
    inp_o[...] = inp
    hm_o[...] = jnp.dot(jnp.maximum(inp, 0.0), wh[...],
                        preferred_element_type=jnp.float32)


def _tc_mm0(f_bonds, W_i, W_h):
    nb = f_bonds.shape[0]
    B = 4000
    return pl.pallas_call(
        _mm0_body,
        grid=(nb // B,),
        in_specs=[
            pl.BlockSpec((B, f_bonds.shape[1]), lambda i: (i, 0)),
            pl.BlockSpec(W_i.shape, lambda i: (0, 0)),
            pl.BlockSpec(W_h.shape, lambda i: (0, 0)),
        ],
        out_specs=[
            pl.BlockSpec((B, H), lambda i: (i, 0)),
            pl.BlockSpec((B, H), lambda i: (i, 0)),
        ],
        out_shape=[
            jax.ShapeDtypeStruct((nb, H), jnp.float32),
            jax.ShapeDtypeStruct((nb, H), jnp.float32),
        ],
    )(f_bonds, W_i, W_h)


def _mm1_body(inp, g, wh, hm_o):
    m = jnp.maximum(inp[...] + g[...], 0.0)
    hm_o[...] = jnp.dot(m, wh[...], preferred_element_type=jnp.float32)


def _tc_mm1(inp, g, W_h, rows=None):
    nb = inp.shape[0] if rows is None else rows
    B = 4000
    return pl.pallas_call(
        _mm1_body,
        grid=(nb // B,),
        in_specs=[
            pl.BlockSpec((B, H), lambda i: (i, 0)),
            pl.BlockSpec((B, H), lambda i: (i, 0)),
            pl.BlockSpec(W_h.shape, lambda i: (0, 0)),
        ],
        out_specs=pl.BlockSpec((B, H), lambda i: (i, 0)),
        out_shape=jax.ShapeDtypeStruct((nb, H), jnp.float32),
    )(inp, g, W_h)


def _sc_gathersum(hm, a2b_flat):
    P = 4
    nrows = a2b_flat.shape[0]
    n_at = nrows // 16
    CA = 8
    RPC = CA * 16
    n_chunks = n_at // CA
    NC = -(-n_chunks // NW)
    NC = -(-NC // P) * P
    SLAB = NC * RPC

    @functools.partial(
        pl.kernel,
        out_type=jax.ShapeDtypeStruct((n_at, H), jnp.float32),
        mesh=_mesh(),
        scratch_types=(
            [pltpu.VMEM((SLAB,), jnp.int32)]
            + [pltpu.VMEM((RPC, H), jnp.float32) for _ in range(P)]
            + [pltpu.VMEM((CA, H), jnp.float32) for _ in range(P)]
            + [pltpu.SemaphoreType.DMA for _ in range(2 * P)]
        ),
    )
    def k(hm_hbm, idx_hbm, out_hbm, idx_s, *bufs):
        rows = bufs[0:P]
        outs = bufs[P:2 * P]
        srs = bufs[2 * P:3 * P]
        sos = bufs[3 * P:4 * P]
        w = _wid()
        c0 = (w * (n_chunks - NC)) // (NW - 1)

        pltpu.sync_copy(idx_hbm.at[pl.ds(c0 * RPC, SLAB)], idx_s)
        for p in range(P):
            pltpu.async_copy(
                hm_hbm.at[idx_s.at[pl.ds(p * RPC, RPC)]], rows[p], srs[p])

        def body(cg, _):
            for p in range(P):
                ci = P * cg + p
                pltpu.make_async_copy(
                    hm_hbm.at[idx_s.at[pl.ds(0, RPC)]], rows[p],
                    srs[p]).wait()

                @pl.when(cg > 0)
                def _():
                    pltpu.make_async_copy(
                        outs[p], out_hbm.at[pl.ds(0, CA)], sos[p]).wait()

                for a in range(CA):
                    for s in range(H // LANES):
                        sl = pl.ds(s * LANES, LANES)
                        acc = rows[p][a * 16, sl]
                        for j in range(1, 16):
                            acc = acc + rows[p][a * 16 + j, sl]
                        outs[p][a, sl] = acc
                pltpu.async_copy(
                    outs[p], out_hbm.at[pl.ds((c0 + ci) * CA, CA)], sos[p])

                @pl.when(ci + P < NC)
                def _():
                    pltpu.async_copy(
                        hm_hbm.at[idx_s.at[pl.ds((ci + P) * RPC, RPC)]],
                        rows[p], srs[p])
            return 0

        lax.fori_loop(0, NC // P, body, 0)
        for p in range(P):
            pltpu.make_async_copy(
                outs[p], out_hbm.at[pl.ds(0, CA)], sos[p]).wait()

    return k(hm, a2b_flat)


def _sc_combine(ahm, hm, b2a, b2revb, inp=None, lo=0, hi=None):
    nb = hm.shape[0]
    if hi is None:
        hi = nb
    span = hi - lo
    per_w = span // NW
    R = 64
    P = 2
    n_full = per_w // R
    NC = -(-(n_full + 1) // P) * P
    with_inp = inp is not None

    scratch = (
        [pltpu.VMEM((per_w,), jnp.int32) for _ in range(2)]
        + [pltpu.VMEM((R, H), jnp.float32) for _ in range(3 * P)]
        + [pltpu.SemaphoreType.DMA for _ in range(2 * P)]
    )
    if with_inp:
        scratch += (
            [pltpu.VMEM((R, H), jnp.float32) for _ in range(P)]
            + [pltpu.SemaphoreType.DMA for _ in range(P)]
        )

    def body(ahm_hbm, hm_hbm, b2a_hbm, b2revb_hbm, *rest):
        if with_inp:
            inp_hbm, out_hbm = rest[0], rest[1]
            bufs = rest[2:]
        else:
            out_hbm = rest[0]
            bufs = rest[1:]
        ia_s, ib_s = bufs[0], bufs[1]
        ras = bufs[2:2 + P]
        rbs = bufs[2 + P:2 + 2 * P]
        ous = bufs[2 + 2 * P:2 + 3 * P]
        srs = bufs[2 + 3 * P:2 + 4 * P]
        sos = bufs[2 + 4 * P:2 + 5 * P]
        if with_inp:
            ris = bufs[2 + 5 * P:2 + 6 * P]
            sis = bufs[2 + 6 * P:2 + 7 * P]
        w = _wid()
        b0 = lo + w * per_w

        def cstart(ci):
            return jnp.where(ci < n_full, ci * R, per_w - (NC - ci) * R)

        def fetch(ci, p):
            st = cstart(ci)
            pltpu.async_copy(ahm_hbm.at[ib_s.at[pl.ds(st, R)]],
                             ras[p], srs[p])
            pltpu.async_copy(hm_hbm.at[ia_s.at[pl.ds(st, R)]],
                             rbs[p], srs[p])
            if with_inp:
                pltpu.async_copy(inp_hbm.at[pl.ds(b0 + st, R)],
                                 ris[p], sis[p])

        pltpu.sync_copy(b2a_hbm.at[pl.ds(b0, per_w)], ib_s)
        pltpu.sync_copy(b2revb_hbm.at[pl.ds(b0, per_w)], ia_s)
        for p in range(P):
            fetch(p, p)

        def chunk(cg, _):
            for p in range(P):
                ci = P * cg + p
                st = cstart(ci)
                for _ in range(2):
                    pltpu.make_async_copy(
                        hm_hbm.at[ia_s.at[pl.ds(0, R)]], rbs[p],
                        srs[p]).wait()
                if with_inp:
                    pltpu.make_async_copy(
                        inp_hbm.at[pl.ds(0, R)], ris[p], sis[p]).wait()

                @pl.when(cg > 0)
                def _():
                    pltpu.make_async_copy(
                        ous[p], out_hbm.at[pl.ds(0, R)], sos[p]).wait()

                def row(r, _):
                    for s in range(H // LANES):
                        sl = pl.ds(s * LANES, LANES)
                        x = ras[p][r, sl] - rbs[p][r, sl]
                        if with_inp:
                            x = jnp.maximum(x + ris[p][r, sl], 0.0)
                        ous[p][r, sl] = x
                    return 0

                lax.fori_loop(0, R, row, 0)
                pltpu.async_copy(
                    ous[p], out_hbm.at[pl.ds(b0 + st, R)], sos[p])

                @pl.when(ci + P < NC)
                def _():
                    fetch(ci + P, p)
            return 0

        lax.fori_loop(0, NC // P, chunk, 0)
        for p in range(P):
            pltpu.make_async_copy(
                ous[p], out_hbm.at[pl.ds(0, R)], sos[p]).wait()

    kern = functools.partial(
        pl.kernel,
        out_type=jax.ShapeDtypeStruct((span, H), jnp.float32),
        mesh=_mesh(),
        scratch_types=scratch,
    )(body)
    if with_inp:
        return kern(ahm, hm, b2a, b2revb, inp)
    return kern(ahm, hm, b2a, b2revb)


def _sc_final(inp, ahm, hm, a2b_flat, b2a, b2revb):
    P = 2
    nrows = a2b_flat.shape[0]
    n_at = nrows // 16
    CA = 4
    RPC = CA * 16
    n_chunks = n_at // CA
    NC = -(-n_chunks // NW)
    NC = -(-NC // P) * P
    SLAB = NC * RPC

    @functools.partial(
        pl.kernel,
        out_type=jax.ShapeDtypeStruct((n_at, H), jnp.float32),
        mesh=_mesh(),
        scratch_types=(
            [pltpu.VMEM((SLAB,), jnp.int32)]
            + [pltpu.VMEM((RPC,), jnp.int32) for _ in range(2 * P)]
            + [pltpu.VMEM((RPC, H), jnp.float32) for _ in range(3 * P)]
            + [pltpu.VMEM((CA, H), jnp.float32) for _ in range(P)]
            + [pltpu.SemaphoreType.DMA for _ in range(3 * P)]
        ),
    )
    def k(inp_hbm, ahm_hbm, hm_hbm, a2b_hbm, b2a_hbm, b2revb_hbm, out_hbm,
          slab, *bufs):
        ias = bufs[0:P]
        ibs = bufs[P:2 * P]
        ris = bufs[2 * P:3 * P]
        ras = bufs[3 * P:4 * P]
        rbs = bufs[4 * P:5 * P]
        ous = bufs[5 * P:6 * P]
        sis = bufs[6 * P:7 * P]
        srs = bufs[7 * P:8 * P]
        sos = bufs[8 * P:9 * P]
        w = _wid()
        c0 = (w * (n_chunks - NC)) // (NW - 1)

        pltpu.sync_copy(a2b_hbm.at[pl.ds(c0 * RPC, SLAB)], slab)

        def fetch_idx(ci, p):
            sl = slab.at[pl.ds(ci * RPC, RPC)]
            pltpu.async_copy(b2a_hbm.at[sl], ias[p], sis[p])
            pltpu.async_copy(b2revb_hbm.at[sl], ibs[p], sis[p])

        def fetch_rows(ci, p):
            pltpu.async_copy(inp_hbm.at[slab.at[pl.ds(ci * RPC, RPC)]],
                             ris[p], srs[p])
            pltpu.async_copy(ahm_hbm.at[ias[p]], ras[p], srs[p])
            pltpu.async_copy(hm_hbm.at[ibs[p]], rbs[p], srs[p])

        def wait_idx(p):
            for _ in range(2):
                pltpu.make_async_copy(b2a_hbm.at[slab.at[pl.ds(0, RPC)]],
                                      ias[p], sis[p]).wait()

        def wait_rows(p):
            for _ in range(3):
                pltpu.make_async_copy(hm_hbm.at[ias[p]], rbs[p],
                                      srs[p]).wait()

        fetch_idx(0, 0)
        fetch_idx(1, 1)
        wait_idx(0)
        fetch_rows(0, 0)

        def body(cg, _):
            for p in range(P):
                c = P * cg + p
                pn = (p + 1) % P

                @pl.when(c + 1 < NC)
                def _():
                    wait_idx(pn)
                    fetch_rows(c + 1, pn)

                wait_rows(p)

                @pl.when(c + 2 < NC)
                def _():
                    fetch_idx(c + 2, p)

                @pl.when(cg > 0)
                def _():
                    pltpu.make_async_copy(
                        ous[p], out_hbm.at[pl.ds(0, CA)], sos[p]).wait()

                for a in range(CA):
                    for s in range(H // LANES):
                        sl = pl.ds(s * LANES, LANES)
                        r0 = a * 16
                        acc = jnp.maximum(
                            ris[p][r0, sl] + ras[p][r0, sl] - rbs[p][r0, sl],
                            0.0)
                        for j in range(1, 16):
                            r = r0 + j
                            acc = acc + jnp.maximum(
                                ris[p][r, sl] + ras[p][r, sl] - rbs[p][r, sl],
                                0.0)
                        ous[p][a, sl] = acc
                pltpu.async_copy(
                    ous[p], out_hbm.at[pl.ds((c0 + c) * CA, CA)], sos[p])
            return 0

        lax.fori_loop(0, NC // P, body, 0)
        for p in range(P):
            pltpu.make_async_copy(
                ous[p], out_hbm.at[pl.ds(0, CA)], sos[p]).wait()

    return k(inp, ahm, hm, a2b_flat, b2a, b2revb)


def _readout_body(fa, amr, segr, woa, wom, bor, out_ref, acc, cnt):
    i = pl.program_id(0)
    npg = pl.num_programs(0)

    @pl.when(i == 0)
    def _():
        acc[...] = jnp.zeros_like(acc)
        cnt[...] = jnp.zeros_like(cnt)

    ah = jnp.dot(fa[...], woa[...], preferred_element_type=jnp.float32)
    ah = ah + jnp.dot(amr[...], wom[...], preferred_element_type=jnp.float32)
    ah = jnp.maximum(ah + bor[...], 0.0)

    s = segr[0]
    A = s.shape[1]
    n_mols = acc.shape[0]
    MC = 500
    for h in range(n_mols // MC):
        iota = lax.broadcasted_iota(jnp.int32, (MC, A), 0) + h * MC
        ohT = (jnp.broadcast_to(s, (MC, A)) == iota).astype(jnp.float32)
        acc[pl.ds(h * MC, MC), :] += jnp.dot(
            ohT, ah, preferred_element_type=jnp.float32)
        cnt[pl.ds(h * MC, MC), :] += jnp.sum(ohT, axis=1, keepdims=True)

    @pl.when(i == npg - 1)
    def _():
        out_ref[...] = acc[...] / jnp.maximum(cnt[...], 1.0)


def _tc_readout(f_atoms, am, seg, W_o, b_o, n_mols=2000):
    na = f_atoms.shape[0]
    A = 1000
    seg3 = seg.reshape(na // A, 1, A)
    woa = W_o[:H]
    wom = W_o[H:]
    bor = b_o.reshape(1, H)
    return pl.pallas_call(
        _readout_body,
        grid=(na // A,),
        in_specs=[
            pl.BlockSpec((A, H), lambda i: (i, 0)),
            pl.BlockSpec((A, H), lambda i: (i, 0)),
            pl.BlockSpec((1, 1, A), lambda i: (i, 0, 0)),
            pl.BlockSpec((H, H), lambda i: (0, 0)),
            pl.BlockSpec((H, H), lambda i: (0, 0)),
            pl.BlockSpec((1, H), lambda i: (0, 0)),
        ],
        out_specs=pl.BlockSpec((n_mols, H), lambda i: (0, 0)),
        out_shape=jax.ShapeDtypeStruct((n_mols, H), jnp.float32),
        scratch_shapes=[
            pltpu.VMEM((n_mols, H), jnp.float32),
            pltpu.VMEM((n_mols, 1), jnp.float32),
        ],
        compiler_params=pltpu.CompilerParams(
            dimension_semantics=("arbitrary",)),
    )(f_atoms, am, seg3, woa, wom, bor)


def kernel(f_atoms, f_bonds, a2b, b2a, b2revb, seg, W_i, W_h, W_o, b_o):
    a2b_flat = a2b.reshape(-1)
    nb = f_bonds.shape[0]
    half = nb // 2

    inp, hm = _tc_mm0(f_bonds, W_i, W_h)
    ahm0 = _sc_gathersum(hm, a2b_flat)
    g0 = _sc_combine(ahm0, hm, b2a, b2revb)
    hm1 = _tc_mm1(inp, g0, W_h)
    ahm1 = _sc_gathersum(hm1, a2b_flat)
    msg2 = _sc_combine(ahm1, hm1, b2a, b2revb, inp=inp)
    am2 = _sc_gathersum(msg2, a2b_flat)
    return _tc_readout(f_atoms, am2, seg, W_o, b_o)


# speedup vs baseline: 1.0816x; 1.0782x over previous
import functools

import jax
import jax.numpy as jnp
from jax import lax
from jax.experimental import pallas as pl
from jax.experimental.pallas import tpu as pltpu
from jax.experimental.pallas import tpu_sc as plsc

H = 128
NW = 32
LANES = 16


def _ptree(vals):
    vals = list(vals)
    while len(vals) > 1:
        nxt = [a + b for a, b in zip(vals[0::2], vals[1::2])]
        if len(vals) % 2:
            nxt.append(vals[-1])
        vals = nxt
    return vals[0]


def _wid():
    return lax.axis_index("s") * 2 + lax.axis_index("c")


def _mesh():
    return plsc.VectorSubcoreMesh(core_axis_name="c", subcore_axis_name="s")


def _mm0_body(fb, wi, wh, inp_o, hm_o):
    inp = jnp.dot(fb[...], wi[...], preferred_element_type=jnp.float32)
    inp_o[...] = inp
    hm_o[...] = jnp.dot(jnp.maximum(inp, 0.0), wh[...],
                        preferred_element_type=jnp.float32)


def _tc_mm0(f_bonds, W_i, W_h):
    nb = f_bonds.shape[0]
    B = 4000
    return pl.pallas_call(
        _mm0_body,
        grid=(nb // B,),
        in_specs=[
            pl.BlockSpec((B, f_bonds.shape[1]), lambda i: (i, 0)),
            pl.BlockSpec(W_i.shape, lambda i: (0, 0)),
            pl.BlockSpec(W_h.shape, lambda i: (0, 0)),
        ],
        out_specs=[
            pl.BlockSpec((B, H), lambda i: (i, 0)),
            pl.BlockSpec((B, H), lambda i: (i, 0)),
        ],
        out_shape=[
            jax.ShapeDtypeStruct((nb, H), jnp.float32),
            jax.ShapeDtypeStruct((nb, H), jnp.float32),
        ],
    )(f_bonds, W_i, W_h)


def _mm1_body(inp, g, wh, hm_o):
    m = jnp.maximum(inp[...] + g[...], 0.0)
    hm_o[...] = jnp.dot(m, wh[...], preferred_element_type=jnp.float32)


def _mm1_body_dst(inp, g, wh, dst, hm_o):
    _mm1_body(inp, g, wh, hm_o)


def _tc_mm1(inp, g, W_h, lo=0, hi=None, dst=None):
    nb = inp.shape[0]
    if hi is None:
        hi = nb
    B = 4000
    j0 = lo // B
    in_specs = [
        pl.BlockSpec((B, H), lambda i: (i + j0, 0)),
        pl.BlockSpec((B, H), lambda i: (i + j0, 0)),
        pl.BlockSpec(W_h.shape, lambda i: (0, 0)),
    ]
    args = [inp, g, W_h]
    aliases = {}
    if dst is not None:
        in_specs.append(pl.BlockSpec(memory_space=pl.ANY))
        args.append(dst)
        aliases = {3: 0}
    return pl.pallas_call(
        _mm1_body if dst is None else _mm1_body_dst,
        grid=((hi - lo) // B,),
        in_specs=in_specs,
        out_specs=pl.BlockSpec((B, H), lambda i: (i + j0, 0)),
        out_shape=jax.ShapeDtypeStruct((nb, H), jnp.float32),
        input_output_aliases=aliases,
    )(*args)


def _sc_gathersum(hm, a2b_flat):
    P = 2
    nrows = a2b_flat.shape[0]
    n_at = nrows // 16
    CA = 8
    RPC = CA * 16
    n_chunks = n_at // CA
    NC = -(-n_chunks // NW)
    NC = -(-NC // P) * P
    SLAB = NC * RPC

    @functools.partial(
        pl.kernel,
        out_type=jax.ShapeDtypeStruct((n_at, H), jnp.float32),
        mesh=_mesh(),
        scratch_types=(
            [pltpu.VMEM((SLAB,), jnp.int32)]
            + [pltpu.VMEM((RPC, H), jnp.float32) for _ in range(P)]
            + [pltpu.VMEM((CA, H), jnp.float32) for _ in range(P)]
            + [pltpu.SemaphoreType.DMA for _ in range(2 * P)]
        ),
    )
    def k(hm_hbm, idx_hbm, out_hbm, idx_s, *bufs):
        rows = bufs[0:P]
        outs = bufs[P:2 * P]
        srs = bufs[2 * P:3 * P]
        sos = bufs[3 * P:4 * P]
        w = _wid()
        c0 = (w * (n_chunks - NC)) // (NW - 1)

        pltpu.sync_copy(idx_hbm.at[pl.ds(c0 * RPC, SLAB)], idx_s)
        for p in range(P):
            pltpu.async_copy(
                hm_hbm.at[idx_s.at[pl.ds(p * RPC, RPC)]], rows[p], srs[p])

        def body(cg, _):
            for p in range(P):
                ci = P * cg + p
                pltpu.make_async_copy(
                    hm_hbm.at[idx_s.at[pl.ds(0, RPC)]], rows[p],
                    srs[p]).wait()

                @pl.when(cg > 0)
                def _():
                    pltpu.make_async_copy(
                        outs[p], out_hbm.at[pl.ds(0, CA)], sos[p]).wait()

                for a in range(CA):
                    for s in range(H // LANES):
                        sl = pl.ds(s * LANES, LANES)
                        acc = rows[p][a * 16, sl]
                        for j in range(1, 16):
                            acc = acc + rows[p][a * 16 + j, sl]
                        outs[p][a, sl] = acc
                pltpu.async_copy(
                    outs[p], out_hbm.at[pl.ds((c0 + ci) * CA, CA)], sos[p])

                @pl.when(ci + P < NC)
                def _():
                    pltpu.async_copy(
                        hm_hbm.at[idx_s.at[pl.ds((ci + P) * RPC, RPC)]],
                        rows[p], srs[p])
            return 0

        lax.fori_loop(0, NC // P, body, 0)
        for p in range(P):
            pltpu.make_async_copy(
                outs[p], out_hbm.at[pl.ds(0, CA)], sos[p]).wait()

    return k(hm, a2b_flat)


def _sc_combine(ahm, hm, b2a, b2revb, inp=None, lo=0, hi=None):
    nb = hm.shape[0]
    if hi is None:
        hi = nb
    span = hi - lo
    per_w = (span // NW + 32) & ~7
    R = 64
    P = 2
    n_full = per_w // R
    NC = -(-(n_full + 1) // P) * P
    with_inp = inp is not None

    scratch = (
        [pltpu.VMEM((per_w,), jnp.int32) for _ in range(2)]
        + [pltpu.VMEM((R, H), jnp.float32) for _ in range(3 * P)]
        + [pltpu.SemaphoreType.DMA for _ in range(2 * P)]
    )
    if with_inp:
        scratch += (
            [pltpu.VMEM((R, H), jnp.float32) for _ in range(P)]
            + [pltpu.SemaphoreType.DMA for _ in range(P)]
        )

    def body(ahm_hbm, hm_hbm, b2a_hbm, b2revb_hbm, *rest):
        if with_inp:
            inp_hbm, out_hbm = rest[0], rest[1]
            bufs = rest[2:]
        else:
            out_hbm = rest[0]
            bufs = rest[1:]
        ia_s, ib_s = bufs[0], bufs[1]
        ras = bufs[2:2 + P]
        rbs = bufs[2 + P:2 + 2 * P]
        ous = bufs[2 + 2 * P:2 + 3 * P]
        srs = bufs[2 + 3 * P:2 + 4 * P]
        sos = bufs[2 + 4 * P:2 + 5 * P]
        if with_inp:
            ris = bufs[2 + 5 * P:2 + 6 * P]
            sis = bufs[2 + 6 * P:2 + 7 * P]
        w = _wid()
        b0 = pl.multiple_of(lo + (((w * (span - per_w)) // (NW - 1)) & ~7), 8)

        def cstart(ci):
            return jnp.where(ci < n_full, ci * R, per_w - (NC - ci) * R)

        def fetch(ci, p):
            st = cstart(ci)
            pltpu.async_copy(ahm_hbm.at[ib_s.at[pl.ds(st, R)]],
                             ras[p], srs[p])
            pltpu.async_copy(hm_hbm.at[ia_s.at[pl.ds(st, R)]],
                             rbs[p], srs[p])
            if with_inp:
                pltpu.async_copy(inp_hbm.at[pl.ds(b0 + st, R)],
                                 ris[p], sis[p])

        pltpu.sync_copy(b2a_hbm.at[pl.ds(b0, per_w)], ib_s)
        pltpu.sync_copy(b2revb_hbm.at[pl.ds(b0, per_w)], ia_s)
        for p in range(P):
            fetch(p, p)

        def chunk(cg, _):
            for p in range(P):
                ci = P * cg + p
                st = cstart(ci)
                for _ in range(2):
                    pltpu.make_async_copy(
                        hm_hbm.at[ia_s.at[pl.ds(0, R)]], rbs[p],
                        srs[p]).wait()
                if with_inp:
                    pltpu.make_async_copy(
                        inp_hbm.at[pl.ds(0, R)], ris[p], sis[p]).wait()

                @pl.when(cg > 0)
                def _():
                    pltpu.make_async_copy(
                        ous[p], out_hbm.at[pl.ds(0, R)], sos[p]).wait()

                def row(r, _):
                    for s in range(H // LANES):
                        sl = pl.ds(s * LANES, LANES)
                        x = ras[p][r, sl] - rbs[p][r, sl]
                        if with_inp:
                            x = jnp.maximum(x + ris[p][r, sl], 0.0)
                        ous[p][r, sl] = x
                    return 0

                lax.fori_loop(0, R, row, 0)
                pltpu.async_copy(
                    ous[p], out_hbm.at[pl.ds(b0 + st, R)], sos[p])

                @pl.when(ci + P < NC)
                def _():
                    fetch(ci + P, p)
            return 0

        lax.fori_loop(0, NC // P, chunk, 0)
        for p in range(P):
            pltpu.make_async_copy(
                ous[p], out_hbm.at[pl.ds(0, R)], sos[p]).wait()

    kern = functools.partial(
        pl.kernel,
        out_type=jax.ShapeDtypeStruct((nb, H), jnp.float32),
        mesh=_mesh(),
        scratch_types=scratch,
    )(body)
    if with_inp:
        return kern(ahm, hm, b2a, b2revb, inp)
    return kern(ahm, hm, b2a, b2revb)


def _sc_final(inp, ahm, hm, a2b_flat, b2a, b2revb):
    P = 2
    nrows = a2b_flat.shape[0]
    n_at = nrows // 16
    CA = 4
    RPC = CA * 16
    n_chunks = n_at // CA
    NC = -(-n_chunks // NW)
    NC = -(-NC // P) * P
    SLAB = NC * RPC

    @functools.partial(
        pl.kernel,
        out_type=jax.ShapeDtypeStruct((n_at, H), jnp.float32),
        mesh=_mesh(),
        scratch_types=(
            [pltpu.VMEM((SLAB,), jnp.int32)]
            + [pltpu.VMEM((RPC,), jnp.int32) for _ in range(2 * P)]
            + [pltpu.VMEM((RPC, H), jnp.float32) for _ in range(3 * P)]
            + [pltpu.VMEM((CA, H), jnp.float32) for _ in range(P)]
            + [pltpu.SemaphoreType.DMA for _ in range(3 * P)]
        ),
    )
    def k(inp_hbm, ahm_hbm, hm_hbm, a2b_hbm, b2a_hbm, b2revb_hbm, out_hbm,
          slab, *bufs):
        ias = bufs[0:P]
        ibs = bufs[P:2 * P]
        ris = bufs[2 * P:3 * P]
        ras = bufs[3 * P:4 * P]
        rbs = bufs[4 * P:5 * P]
        ous = bufs[5 * P:6 * P]
        sis = bufs[6 * P:7 * P]
        srs = bufs[7 * P:8 * P]
        sos = bufs[8 * P:9 * P]
        w = _wid()
        c0 = (w * (n_chunks - NC)) // (NW - 1)

        pltpu.sync_copy(a2b_hbm.at[pl.ds(c0 * RPC, SLAB)], slab)

        def fetch_idx(ci, p):
            sl = slab.at[pl.ds(ci * RPC, RPC)]
            pltpu.async_copy(b2a_hbm.at[sl], ias[p], sis[p])
            pltpu.async_copy(b2revb_hbm.at[sl], ibs[p], sis[p])

        def fetch_rows(ci, p):
            pltpu.async_copy(inp_hbm.at[slab.at[pl.ds(ci * RPC, RPC)]],
                             ris[p], srs[p])
            pltpu.async_copy(ahm_hbm.at[ias[p]], ras[p], srs[p])
            pltpu.async_copy(hm_hbm.at[ibs[p]], rbs[p], srs[p])

        def wait_idx(p):
            for _ in range(2):
                pltpu.make_async_copy(b2a_hbm.at[slab.at[pl.ds(0, RPC)]],
                                      ias[p], sis[p]).wait()

        def wait_rows(p):
            for _ in range(3):
                pltpu.make_async_copy(hm_hbm.at[ias[p]], rbs[p],
                                      srs[p]).wait()

        fetch_idx(0, 0)
        fetch_idx(1, 1)
        wait_idx(0)
        fetch_rows(0, 0)

        def body(cg, _):
            for p in range(P):
                c = P * cg + p
                pn = (p + 1) % P

                @pl.when(c + 1 < NC)
                def _():
                    wait_idx(pn)
                    fetch_rows(c + 1, pn)

                wait_rows(p)

                @pl.when(c + 2 < NC)
                def _():
                    fetch_idx(c + 2, p)

                @pl.when(cg > 0)
                def _():
                    pltpu.make_async_copy(
                        ous[p], out_hbm.at[pl.ds(0, CA)], sos[p]).wait()

                for a in range(CA):
                    for s in range(H // LANES):
                        sl = pl.ds(s * LANES, LANES)
                        r0 = a * 16
                        acc = jnp.maximum(
                            ris[p][r0, sl] + ras[p][r0, sl] - rbs[p][r0, sl],
                            0.0)
                        for j in range(1, 16):
                            r = r0 + j
                            acc = acc + jnp.maximum(
                                ris[p][r, sl] + ras[p][r, sl] - rbs[p][r, sl],
                                0.0)
                        ous[p][a, sl] = acc
                pltpu.async_copy(
                    ous[p], out_hbm.at[pl.ds((c0 + c) * CA, CA)], sos[p])
            return 0

        lax.fori_loop(0, NC // P, body, 0)
        for p in range(P):
            pltpu.make_async_copy(
                ous[p], out_hbm.at[pl.ds(0, CA)], sos[p]).wait()

    return k(inp, ahm, hm, a2b_flat, b2a, b2revb)


def _readout_body(fa, amr, segr, woa, wom, bor, out_ref, acc, cnt):
    i = pl.program_id(0)
    npg = pl.num_programs(0)

    @pl.when(i == 0)
    def _():
        acc[...] = jnp.zeros_like(acc)
        cnt[...] = jnp.zeros_like(cnt)

    ah = jnp.dot(fa[...], woa[...], preferred_element_type=jnp.float32)
    ah = ah + jnp.dot(amr[...], wom[...], preferred_element_type=jnp.float32)
    ah = jnp.maximum(ah + bor[...], 0.0)

    s = segr[0]
    A = s.shape[1]
    n_mols = acc.shape[0]
    MC = 500
    for h in range(n_mols // MC):
        iota = lax.broadcasted_iota(jnp.int32, (MC, A), 0) + h * MC
        ohT = (jnp.broadcast_to(s, (MC, A)) == iota).astype(jnp.float32)
        acc[pl.ds(h * MC, MC), :] += jnp.dot(
            ohT, ah, preferred_element_type=jnp.float32)
        cnt[pl.ds(h * MC, MC), :] += jnp.sum(ohT, axis=1, keepdims=True)

    @pl.when(i == npg - 1)
    def _():
        out_ref[...] = acc[...] / jnp.maximum(cnt[...], 1.0)


def _tc_readout(f_atoms, am, seg, W_o, b_o, n_mols=2000):
    na = f_atoms.shape[0]
    A = 1000
    seg3 = seg.reshape(na // A, 1, A)
    woa = W_o[:H]
    wom = W_o[H:]
    bor = b_o.reshape(1, H)
    return pl.pallas_call(
        _readout_body,
        grid=(na // A,),
        in_specs=[
            pl.BlockSpec((A, H), lambda i: (i, 0)),
            pl.BlockSpec((A, H), lambda i: (i, 0)),
            pl.BlockSpec((1, 1, A), lambda i: (i, 0, 0)),
            pl.BlockSpec((H, H), lambda i: (0, 0)),
            pl.BlockSpec((H, H), lambda i: (0, 0)),
            pl.BlockSpec((1, H), lambda i: (0, 0)),
        ],
        out_specs=pl.BlockSpec((n_mols, H), lambda i: (0, 0)),
        out_shape=jax.ShapeDtypeStruct((n_mols, H), jnp.float32),
        scratch_shapes=[
            pltpu.VMEM((n_mols, H), jnp.float32),
            pltpu.VMEM((n_mols, 1), jnp.float32),
        ],
        compiler_params=pltpu.CompilerParams(
            dimension_semantics=("arbitrary",)),
    )(f_atoms, am, seg3, woa, wom, bor)


def kernel(f_atoms, f_bonds, a2b, b2a, b2revb, seg, W_i, W_h, W_o, b_o):
    a2b_flat = a2b.reshape(-1)
    nb = f_bonds.shape[0]
    half = nb // 2

    inp, hm = _tc_mm0(f_bonds, W_i, W_h)
    ahm0 = _sc_gathersum(hm, a2b_flat)
    g0a = _sc_combine(ahm0, hm, b2a, b2revb, lo=0, hi=half)
    g0b = _sc_combine(ahm0, hm, b2a, b2revb, lo=half, hi=nb)
    h1a = _tc_mm1(inp, g0a, W_h, lo=0, hi=half)
    hm1 = _tc_mm1(inp, g0b, W_h, lo=half, hi=nb, dst=h1a)
    ahm1 = _sc_gathersum(hm1, a2b_flat)
    msg2 = _sc_combine(ahm1, hm1, b2a, b2revb, inp=inp)
    am2 = _sc_gathersum(msg2, a2b_flat)
    return _tc_readout(f_atoms, am2, seg, W_o, b_o)
